# baseline (speedup 1.0000x reference)
import functools

import jax
import jax.numpy as jnp
from jax import lax
from jax.experimental import pallas as pl
from jax.experimental.pallas import tpu as pltpu
from jax.experimental.pallas import tpu_sc as plsc

D_MODEL = 1024
SEQ = 8192
BATCH = 4
NC = 2
NS = 16
NW = NC * NS
LANES = 16
ROWS_PER_W = SEQ // NW
CHUNK = 8
NCHUNKS = ROWS_PER_W // CHUNK
VECS_PER_CHUNK = CHUNK * D_MODEL // LANES


def _sc_body(a_hbm, x_hbm, pt_hbm, pe_hbm, out_hbm,
             a_v, pt_v, pe_v, bl_v,
             x_va, x_vb, o_v0, o_v1, o_v2, o_v3,
             pt_sem, pe_sem, x_sema, x_semb,
             o_sem0, o_sem1, o_sem2, o_sem3):
    cid = lax.axis_index("c")
    sid = lax.axis_index("s")
    wid = sid * NC + cid

    pltpu.sync_copy(a_hbm, a_v)
    t = a_v[...]
    a = 1.0 / (1.0 + jnp.exp(-t))
    b_coef = 1.0 - a

    base_row = wid * ROWS_PER_W
    x_bufs = (x_va, x_vb)
    x_sems = (x_sema, x_semb)
    o_bufs = (o_v0, o_v1, o_v2, o_v3)
    o_sems = (o_sem0, o_sem1, o_sem2, o_sem3)

    def tables_copy(row0):
        return (
            pltpu.make_async_copy(pt_hbm.at[pl.ds(row0, CHUNK)], pt_v, pt_sem),
            pltpu.make_async_copy(pe_hbm.at[pl.ds(row0, CHUNK)], pe_v, pe_sem),
        )

    def x_copy(row0, p):
        return pltpu.make_async_copy(
            x_hbm.at[:, pl.ds(row0, CHUNK)], x_bufs[p], x_sems[p])

    def o_copy(b, row0):
        return pltpu.make_async_copy(
            o_bufs[b], out_hbm.at[b, pl.ds(row0, CHUNK)], o_sems[b])

    for cp in tables_copy(base_row):
        cp.start()
    x_copy(base_row, 0).start()
    x_copy(base_row + CHUNK, 1).start()

    def half_chunk(c, p):
        row0 = base_row + c * CHUNK
        x_v = x_bufs[p]

        for cp in tables_copy(row0):
            cp.wait()
        x_copy(row0, p).wait()


        @pl.when(c > 0)
        def _():
            o_copy(0, row0 - CHUNK).wait()

        @plsc.parallel_loop(0, VECS_PER_CHUNK, unroll=8)
        def _blend(k):
            r = k >> 6
            idx = pl.ds((k & 63) * LANES, LANES)
            bl = a * pt_v[r, idx] + b_coef * pe_v[r, idx]
            bl_v[r, idx] = bl
            o_v0[r, idx] = x_v[0, r, idx] + bl

        @pl.when(c < NCHUNKS - 1)
        def _():
            for cp in tables_copy(row0 + CHUNK):
                cp.start()
        o_copy(0, row0).start()

        for b in range(1, BATCH):
            o_v = o_bufs[b]

            @pl.when(c > 0)
            def _(b=b):
                o_copy(b, row0 - CHUNK).wait()

            @plsc.parallel_loop(0, VECS_PER_CHUNK, unroll=8)
            def _add(k, b=b, o_v=o_v):
                r = k >> 6
                idx = pl.ds((k & 63) * LANES, LANES)
                o_v[r, idx] = x_v[b, r, idx] + bl_v[r, idx]

            o_copy(b, row0).start()

        @pl.when(c < NCHUNKS - 2)
        def _():
            x_copy(row0 + 2 * CHUNK, p).start()

    def pair_body(i, carry):
        c = i * 2
        half_chunk(c, 0)
        half_chunk(c + 1, 1)
        return carry

    lax.fori_loop(0, NCHUNKS // 2, pair_body, 0)

    last_row0 = base_row + (NCHUNKS - 1) * CHUNK
    for b in range(BATCH):
        o_copy(b, last_row0).wait()


def kernel(x, pos_table, alpha, pe):
    batch, seq_len, d_model = x.shape
    a16 = jnp.broadcast_to(jnp.reshape(alpha, (1,)), (LANES,)).astype(jnp.float32)
    pt = pos_table[:seq_len]
    fpe = pe[:seq_len]

    mesh = plsc.VectorSubcoreMesh(core_axis_name="c", subcore_axis_name="s")
    sck = functools.partial(
        pl.kernel,
        out_type=jax.ShapeDtypeStruct((batch, seq_len, d_model), jnp.float32),
        mesh=mesh,
        scratch_types=(
            [pltpu.VMEM((LANES,), jnp.float32)]
            + [pltpu.VMEM((CHUNK, D_MODEL), jnp.float32)] * 3
            + [pltpu.VMEM((BATCH, CHUNK, D_MODEL), jnp.float32)] * 2
            + [pltpu.VMEM((CHUNK, D_MODEL), jnp.float32)] * 4
            + [pltpu.SemaphoreType.DMA] * 8
        ),
    )(_sc_body)
    return sck(a16, x, pt, fpe)


# speedup vs baseline: 1.0042x; 1.0042x over previous
import functools

import jax
import jax.numpy as jnp
from jax import lax
from jax.experimental import pallas as pl
from jax.experimental.pallas import tpu as pltpu
from jax.experimental.pallas import tpu_sc as plsc

D_MODEL = 1024
SEQ = 8192
BATCH = 4
NC = 2
NS = 16
NW = NC * NS
LANES = 16
ROWS_PER_W = SEQ // NW
CHUNK = 8
NCHUNKS = ROWS_PER_W // CHUNK
VECS_PER_CHUNK = CHUNK * D_MODEL // LANES


def _sc_body(a_hbm, x_hbm, pt_hbm, pe_hbm, out_hbm,
             a_v, pt_v, pe_v, bl_v,
             x_v0, x_v1, x_v2, x_v3, o_v0, o_v1, o_v2, o_v3,
             pt_sem, pe_sem, x_sem0, x_sem1, x_sem2, x_sem3,
             o_sem0, o_sem1, o_sem2, o_sem3):
    cid = lax.axis_index("c")
    sid = lax.axis_index("s")
    wid = sid * NC + cid

    pltpu.sync_copy(a_hbm, a_v)
    t = a_v[...]
    a = 1.0 / (1.0 + jnp.exp(-t))
    b_coef = 1.0 - a

    base_row = wid * ROWS_PER_W
    x_bufs = (x_v0, x_v1, x_v2, x_v3)
    o_bufs = (o_v0, o_v1, o_v2, o_v3)
    x_sems = (x_sem0, x_sem1, x_sem2, x_sem3)
    o_sems = (o_sem0, o_sem1, o_sem2, o_sem3)

    def tables_copy(row0):
        return (
            pltpu.make_async_copy(pt_hbm.at[pl.ds(row0, CHUNK)], pt_v, pt_sem),
            pltpu.make_async_copy(pe_hbm.at[pl.ds(row0, CHUNK)], pe_v, pe_sem),
        )

    def x_copy(b, row0):
        return pltpu.make_async_copy(
            x_hbm.at[b, pl.ds(row0, CHUNK)], x_bufs[b], x_sems[b])

    def o_copy(b, row0):
        return pltpu.make_async_copy(
            o_bufs[b], out_hbm.at[b, pl.ds(row0, CHUNK)], o_sems[b])

    for cp in tables_copy(base_row):
        cp.start()
    for b in range(BATCH):
        x_copy(b, base_row).start()

    def chunk_body(c, carry):
        row0 = base_row + c * CHUNK
        nrow0 = row0 + CHUNK

        for cp in tables_copy(row0):
            cp.wait()
        x_copy(0, row0).wait()

        @pl.when(c > 0)
        def _():
            o_copy(0, row0 - CHUNK).wait()

        @plsc.parallel_loop(0, VECS_PER_CHUNK, unroll=8)
        def _blend(k):
            r = k >> 6
            idx = pl.ds((k & 63) * LANES, LANES)
            bl = a * pt_v[r, idx] + b_coef * pe_v[r, idx]
            bl_v[r, idx] = bl
            o_v0[r, idx] = x_v0[r, idx] + bl

        @pl.when(c < NCHUNKS - 1)
        def _():
            for cp in tables_copy(nrow0):
                cp.start()
        o_copy(0, row0).start()

        @pl.when(c < NCHUNKS - 1)
        def _():
            x_copy(0, nrow0).start()

        for b, (x_v, o_v) in enumerate(zip(x_bufs, o_bufs)):
            if b == 0:
                continue

            x_copy(b, row0).wait()

            @pl.when(c > 0)
            def _(b=b):
                o_copy(b, row0 - CHUNK).wait()

            @plsc.parallel_loop(0, VECS_PER_CHUNK, unroll=8)
            def _add(k, x_v=x_v, o_v=o_v):
                r = k >> 6
                idx = pl.ds((k & 63) * LANES, LANES)
                o_v[r, idx] = x_v[r, idx] + bl_v[r, idx]

            o_copy(b, row0).start()

            @pl.when(c < NCHUNKS - 1)
            def _(b=b):
                x_copy(b, nrow0).start()

        return carry

    lax.fori_loop(0, NCHUNKS, chunk_body, 0)

    last_row0 = base_row + (NCHUNKS - 1) * CHUNK
    for b in range(BATCH):
        o_copy(b, last_row0).wait()


def kernel(x, pos_table, alpha, pe):
    batch, seq_len, d_model = x.shape
    a16 = jnp.broadcast_to(jnp.reshape(alpha, (1,)), (LANES,)).astype(jnp.float32)
    pt = pos_table[:seq_len]
    fpe = pe[:seq_len]

    mesh = plsc.VectorSubcoreMesh(core_axis_name="c", subcore_axis_name="s")
    sck = functools.partial(
        pl.kernel,
        out_type=jax.ShapeDtypeStruct((batch, seq_len, d_model), jnp.float32),
        mesh=mesh,
        scratch_types=(
            [pltpu.VMEM((LANES,), jnp.float32)]
            + [pltpu.VMEM((CHUNK, D_MODEL), jnp.float32)] * 11
            + [pltpu.SemaphoreType.DMA] * 10
        ),
    )(_sc_body)
    return sck(a16, x, pt, fpe)
